# Initial kernel scaffold; baseline (speedup 1.0000x reference)
import jax
import jax.numpy as jnp
from jax.experimental import pallas as pl


def kernel(x, W):
    raise NotImplementedError("write your pallas kernel here")


# speedup vs baseline: 2.2180x; 2.2180x over previous
import functools

import jax
import jax.numpy as jnp
from jax import lax
from jax.experimental import pallas as pl
from jax.experimental.pallas import tpu as pltpu
from jax.experimental.pallas import tpu_sc as plsc

NC, NS, L = 2, 16, 16
NW = NC * NS
ROWS, COLS, D = 16384, 1024, 256
K = 8
RB = 8
NCHUNK = COLS // L
RPW = ROWS // NW
NBLK = RPW // RB
GIDX = RB * K
IDXPAD = GIDX + K

_mesh = plsc.VectorSubcoreMesh(core_axis_name="c", subcore_axis_name="s")


@functools.partial(
    pl.kernel,
    out_type=jax.ShapeDtypeStruct((ROWS, D), jnp.float32),
    mesh=_mesh,
    scratch_types=[
        pltpu.VMEM((RB, COLS), jnp.float32),
        pltpu.VMEM((IDXPAD,), jnp.int32),
        pltpu.VMEM((IDXPAD, D), jnp.float32),
        pltpu.VMEM((RB, D), jnp.float32),
        pltpu.SemaphoreType.DMA,
    ],
    compiler_params=pltpu.CompilerParams(needs_layout_passes=False),
)
def _nkq_sc(x_hbm, wt_hbm, out_hbm, x_v, idx_v, rows_v, out_v, sem):
    wid = lax.axis_index("s") * NC + lax.axis_index("c")
    base0 = wid * RPW
    lanes = lax.iota(jnp.int32, L)
    store_mask = lanes < K
    neg_inf = jnp.full((L,), -jnp.inf, dtype=jnp.float32)
    zeros_i = jnp.zeros((L,), dtype=jnp.int32)

    idx_v[pl.ds(IDXPAD - L, L)] = zeros_i

    def block_body(b, carry):
        rowbase = base0 + b * RB
        pltpu.sync_copy(x_hbm.at[pl.ds(rowbase, RB)], x_v)

        def chunk_body(c, st):
            colv = lanes + c * L
            new = []
            for r in range(RB):
                rk, rv = st[2 * r], st[2 * r + 1]
                ck = x_v[r, pl.ds(c * L, L)]
                sk, sv = plsc.sort_key_val(ck, colv, descending=False)
                m = rk >= sk
                mk = jnp.where(m, rk, sk)
                mv = jnp.where(m, rv, sv)
                rk, rv = plsc.sort_key_val(mk, mv, descending=True)
                new += [rk, rv]
            return tuple(new)

        init = (neg_inf, zeros_i) * RB
        fin = lax.fori_loop(0, NCHUNK, chunk_body, init)
        for r in range(RB):
            plsc.store_compressed(
                idx_v.at[pl.ds(r * K, L)], fin[2 * r + 1], mask=store_mask
            )

        pltpu.async_copy(wt_hbm.at[idx_v], rows_v, sem).wait()

        def acc_body(j, a):
            for r in range(RB):
                s = rows_v[r * K, pl.ds(j * L, L)]
                for k in range(1, K):
                    s = s + rows_v[r * K + k, pl.ds(j * L, L)]
                out_v[r, pl.ds(j * L, L)] = s
            return a

        lax.fori_loop(0, D // L, acc_body, 0)
        pltpu.sync_copy(out_v, out_hbm.at[pl.ds(rowbase, RB)])
        return carry

    lax.fori_loop(0, NBLK, block_body, 0)


def kernel(x, W):
    return _nkq_sc(x, W.T)
